# Initial kernel scaffold; baseline (speedup 1.0000x reference)
import jax
import jax.numpy as jnp
from jax.experimental import pallas as pl


def kernel(x, len_x, embedding, Wf, bf, Wb, bb, Wm, bm):
    raise NotImplementedError("write your pallas kernel here")


# speedup vs baseline: 2.5138x; 2.5138x over previous
import jax
import jax.numpy as jnp
from jax import lax
from jax.experimental import pallas as pl
from jax.experimental.pallas import tpu as pltpu
from jax.experimental.pallas import tpu_sc as plsc

_D = 128
_H = 256
_CT = 64

_NC = 2
_NS = 16
_NW = _NC * _NS


def _gather_body(table_hbm, idx_hbm, out_hbm, idx_v, rows_v, sem):
    wid = lax.axis_index("s") * _NC + lax.axis_index("c")
    bpw = idx_v.shape[0]
    base = wid * bpw
    pltpu.sync_copy(idx_hbm.at[pl.ds(base, bpw)], idx_v)
    pltpu.async_copy(table_hbm.at[idx_v], rows_v, sem).wait()
    pltpu.sync_copy(rows_v, out_hbm.at[pl.ds(base, bpw)])


def _sc_gather(table, idx):
    n = idx.shape[0]
    d = table.shape[1]
    bpw = n // _NW
    mesh = plsc.VectorSubcoreMesh(core_axis_name="c", subcore_axis_name="s")
    k = pl.kernel(
        _gather_body,
        mesh=mesh,
        out_type=jax.ShapeDtypeStruct((n, d), jnp.float32),
        scratch_types=[
            pltpu.VMEM((bpw,), jnp.int32),
            pltpu.VMEM((bpw, d), jnp.float32),
            pltpu.SemaphoreType.DMA,
        ],
    )
    return k(table, idx)


def _bilstm_body(len_ref, embf_ref, embb_ref, wxf_ref, whf_ref, bf_ref,
                 wxb_ref, whb_ref, bb_ref, wm1_ref, wm2_ref,
                 uf_ref, ub_ref,
                 zxf, zxb, hfs, hbs, hf_c, cf_c, hb_c, cb_c):
    g = pl.program_id(0)
    ng = pl.num_programs(0)

    @pl.when(g == 0)
    def _init():
        hf_c[...] = jnp.zeros_like(hf_c)
        cf_c[...] = jnp.zeros_like(cf_c)
        hb_c[...] = jnp.zeros_like(hb_c)
        cb_c[...] = jnp.zeros_like(cb_c)

    ct8 = _CT * 8
    embf = embf_ref[...].reshape(ct8, _D)
    embb = embb_ref[...].reshape(ct8, _D)
    zxf[...] = jnp.dot(embf, wxf_ref[...],
                       preferred_element_type=jnp.float32) + bf_ref[...]
    zxb[...] = jnp.dot(embb, wxb_ref[...],
                       preferred_element_type=jnp.float32) + bb_ref[...]

    whf = whf_ref[...]
    whb = whb_ref[...]
    lens = len_ref[...]
    tb_base = (ng - 1 - g) * _CT

    def step(ct, carry):
        hf, cf, hb, cb = carry

        zf = zxf[pl.ds(ct * 8, 8), :] + jnp.dot(
            hf, whf, preferred_element_type=jnp.float32)
        i_f = jax.nn.sigmoid(zf[:, 0:_H])
        f_f = jax.nn.sigmoid(zf[:, _H:2 * _H])
        g_f = jnp.tanh(zf[:, 2 * _H:3 * _H])
        o_f = jax.nn.sigmoid(zf[:, 3 * _H:4 * _H])
        cf = f_f * cf + i_f * g_f
        hf = o_f * jnp.tanh(cf)
        hfs[ct] = hf

        tb = _CT - 1 - ct
        zb = zxb[pl.ds(tb * 8, 8), :] + jnp.dot(
            hb, whb, preferred_element_type=jnp.float32)
        i_b = jax.nn.sigmoid(zb[:, 0:_H])
        f_b = jax.nn.sigmoid(zb[:, _H:2 * _H])
        g_b = jnp.tanh(zb[:, 2 * _H:3 * _H])
        o_b = jax.nn.sigmoid(zb[:, 3 * _H:4 * _H])
        cb_new = f_b * cb + i_b * g_b
        m = (tb_base + tb < lens).astype(jnp.float32)
        hb = m * (o_b * jnp.tanh(cb_new))
        cb = m * cb_new
        hbs[tb] = hb
        return hf, cf, hb, cb

    hf, cf, hb, cb = lax.fori_loop(
        0, _CT, step, (hf_c[...], cf_c[...], hb_c[...], cb_c[...]))
    hf_c[...] = hf
    cf_c[...] = cf
    hb_c[...] = hb
    cb_c[...] = cb

    uf_ref[...] = jnp.sum(hfs[...] * wm1_ref[...], axis=-1)
    ub_ref[...] = jnp.sum(hbs[...] * wm2_ref[...], axis=-1)


def _bilstm(emb_t, len2, wxf, whf, bf2, wxb, whb, bb2, wm1, wm2):
    t = emb_t.shape[0]
    ng = t // _CT
    full = lambda g: (0, 0)
    outs = pl.pallas_call(
        _bilstm_body,
        grid=(ng,),
        in_specs=[
            pl.BlockSpec((8, 1), full),
            pl.BlockSpec((_CT, 8, _D), lambda g: (g, 0, 0)),
            pl.BlockSpec((_CT, 8, _D), lambda g: (ng - 1 - g, 0, 0)),
            pl.BlockSpec((_D, 4 * _H), full),
            pl.BlockSpec((_H, 4 * _H), full),
            pl.BlockSpec((1, 4 * _H), full),
            pl.BlockSpec((_D, 4 * _H), full),
            pl.BlockSpec((_H, 4 * _H), full),
            pl.BlockSpec((1, 4 * _H), full),
            pl.BlockSpec((1, _H), full),
            pl.BlockSpec((1, _H), full),
        ],
        out_specs=[
            pl.BlockSpec((_CT, 8), lambda g: (g, 0)),
            pl.BlockSpec((_CT, 8), lambda g: (ng - 1 - g, 0)),
        ],
        out_shape=[
            jax.ShapeDtypeStruct((t, 8), jnp.float32),
            jax.ShapeDtypeStruct((t, 8), jnp.float32),
        ],
        scratch_shapes=[
            pltpu.VMEM((_CT * 8, 4 * _H), jnp.float32),
            pltpu.VMEM((_CT * 8, 4 * _H), jnp.float32),
            pltpu.VMEM((_CT, 8, _H), jnp.float32),
            pltpu.VMEM((_CT, 8, _H), jnp.float32),
            pltpu.VMEM((8, _H), jnp.float32),
            pltpu.VMEM((8, _H), jnp.float32),
            pltpu.VMEM((8, _H), jnp.float32),
            pltpu.VMEM((8, _H), jnp.float32),
        ],
    )(len2, emb_t, emb_t, wxf, whf, bf2, wxb, whb, bb2, wm1, wm2)
    return outs


def _head_body(uf_ref, ub_ref, bm_ref, len_ref, out_ref):
    tt = lax.broadcasted_iota(jnp.int32, out_ref.shape, 0)
    mask = (tt < len_ref[...]).astype(jnp.float32)
    out_ref[...] = jax.nn.sigmoid(
        uf_ref[...] + ub_ref[...] + bm_ref[...]) * mask


def _head(uf, ub, bm2, len_row):
    t = uf.shape[0]
    return pl.pallas_call(
        _head_body,
        out_shape=jax.ShapeDtypeStruct((t, 8), jnp.float32),
    )(uf, ub, bm2, len_row)


def kernel(x, len_x, embedding, Wf, bf, Wb, bb, Wm, bm):
    b, t = x.shape
    idx = x.T.reshape(-1).astype(jnp.int32)
    emb = _sc_gather(embedding, idx)
    emb_t = emb.reshape(t, b, _D)

    wxf, whf = Wf[:_D], Wf[_D:]
    wxb, whb = Wb[:_D], Wb[_D:]
    wm1 = Wm[:_H, 0].reshape(1, _H)
    wm2 = Wm[_H:, 0].reshape(1, _H)
    len2 = len_x.reshape(b, 1).astype(jnp.int32)
    len_row = len_x.reshape(1, b).astype(jnp.int32)

    uf, ub = _bilstm(emb_t, len2, wxf, whf, bf.reshape(1, -1),
                     wxb, whb, bb.reshape(1, -1), wm1, wm2)
    probs_tb = _head(uf, ub, bm.reshape(1, 1), len_row)
    return probs_tb.T
